# baseline (speedup 1.0000x reference)
import functools

import jax
import jax.numpy as jnp
from jax import lax
from jax.experimental import pallas as pl
from jax.experimental.pallas import tpu as pltpu
from jax.experimental.pallas import tpu_sc as plsc

N = 10000
D = 128
N_CLS = 40
D2 = 48
N_PAD = 10240
N_TILES = 16
ROWS_PER_TILE = N_PAD // N_TILES
CHUNK = 112
E_ORIG = 320000
CHUNKS_PER_TILE = 90
E_PAD = 2 * N_TILES * CHUNKS_PER_TILE * CHUNK
EDGES_PER_CORE = E_PAD // 2
NROW = 3
NIDX = 4
ROWS_BLK = 2000
GRID = N // ROWS_BLK


@functools.lru_cache(maxsize=None)
def _make_scatter(d):
  mesh = plsc.VectorSubcoreMesh(core_axis_name="c", subcore_axis_name="s",
                                num_cores=2, num_subcores=N_TILES)

  @functools.partial(
      pl.kernel,
      out_type=jax.ShapeDtypeStruct((2, N_PAD, d), jnp.float32),
      mesh=mesh,
      compiler_params=pltpu.CompilerParams(use_tc_tiling_on_sc=False),
      scratch_types=(
          [pltpu.VMEM((CHUNK, d), jnp.float32)] * NROW +
          [pltpu.VMEM((CHUNK,), jnp.int32)] * NIDX +
          [pltpu.VMEM((CHUNK,), jnp.int32)] * NIDX +
          [pltpu.VMEM_SHARED((N_PAD, d), jnp.float32)] +
          [pltpu.SemaphoreType.DMA] * (2 * NROW + 2 * NIDX)
      ),
  )
  def scatter_kernel(y_hbm, edge_hbm, tail_hbm, zeros_hbm, out_hbm,
                     rb0, rb1, rb2, si0, si1, si2, si3, di0, di1, di2, di3,
                     acc_sh, gs0, gs1, gs2, cs0, cs1, cs2,
                     ss0, ss1, ss2, ss3, ds0, ds1, ds2, ds3):
    rows = (rb0, rb1, rb2)
    sidx = (si0, si1, si2, si3)
    didx = (di0, di1, di2, di3)
    gsem = (gs0, gs1, gs2)
    csem = (cs0, cs1, cs2)
    ssem = (ss0, ss1, ss2, ss3)
    dsem = (ds0, ds1, ds2, ds3)
    c = lax.axis_index("c")
    s = lax.axis_index("s")
    row0 = s * ROWS_PER_TILE
    tile_id = c * N_TILES + s
    base = tile_id * (CHUNKS_PER_TILE * CHUNK)
    is_last = tile_id == (2 * N_TILES - 1)

    def idx_start(j, t):
      e0 = base + j * CHUNK
      o = j * CHUNK

      @pl.when(is_last)
      def _():
        pltpu.async_copy(tail_hbm.at[0, pl.ds(o, CHUNK)], sidx[t], ssem[t])
        pltpu.async_copy(tail_hbm.at[1, pl.ds(o, CHUNK)], didx[t], dsem[t])

      @pl.when(jnp.logical_not(is_last))
      def _():
        pltpu.async_copy(edge_hbm.at[0, pl.ds(e0, CHUNK)], sidx[t], ssem[t])
        pltpu.async_copy(edge_hbm.at[1, pl.ds(e0, CHUNK)], didx[t], dsem[t])

    def idx_wait(t):
      pltpu.make_async_copy(edge_hbm.at[0, pl.ds(0, CHUNK)], sidx[t],
                            ssem[t]).wait()

    def gather_start(b, t):
      pltpu.async_copy(y_hbm.at[sidx[t]], rows[b], gsem[b])

    def gather_wait(b, t):
      pltpu.make_async_copy(y_hbm.at[sidx[t]], rows[b], gsem[b]).wait()

    def scatter_start(b, t):
      pltpu.make_async_copy(edge_hbm.at[1, pl.ds(0, CHUNK)], didx[t],
                            dsem[t]).wait()
      pltpu.async_copy(rows[b], acc_sh.at[didx[t]], csem[b], add=True)

    def scatter_wait(b, t):
      pltpu.make_async_copy(rows[b], acc_sh.at[didx[t]], csem[b]).wait()

    def step(j, jmod, with_swait=True, with_gather=True, with_idx=True):
      b = jmod % NROW
      t = jmod % NIDX
      gather_wait(b, t)
      if with_swait:
        scatter_wait((jmod - 1) % NROW, (jmod - 1) % NIDX)
      if with_gather:
        idx_wait((jmod + 2) % NIDX)
        gather_start((jmod + 2) % NROW, (jmod + 2) % NIDX)
      scatter_start(b, t)
      if with_idx:
        idx_start(j + 3, (jmod + 3) % NIDX)

    idx_start(0, 0)
    idx_start(1, 1)
    idx_start(2, 2)
    idx_wait(0)
    gather_start(0, 0)
    idx_wait(1)
    gather_start(1, 1)
    pltpu.sync_copy(zeros_hbm, acc_sh.at[pl.ds(row0, ROWS_PER_TILE)])
    plsc.subcore_barrier()

    step(0, 0, with_swait=False)

    def body(i, carry):
      for k in range(12):
        step(12 * i + 1 + k, 1 + k)
      return carry

    lax.fori_loop(0, (CHUNKS_PER_TILE - 6) // 12, body, 0)
    for j in range(CHUNKS_PER_TILE - 5, CHUNKS_PER_TILE):
      step(j, j % 12,
           with_gather=(j <= CHUNKS_PER_TILE - 3),
           with_idx=(j <= CHUNKS_PER_TILE - 4))
    scatter_wait((CHUNKS_PER_TILE - 1) % NROW, (CHUNKS_PER_TILE - 1) % NIDX)
    plsc.subcore_barrier()
    pltpu.sync_copy(acc_sh.at[pl.ds(row0, ROWS_PER_TILE)],
                    out_hbm.at[c, pl.ds(row0, ROWS_PER_TILE)])

  return scatter_kernel


_ALPHA = 0.48870956897735596


def _rows_spec(w):
  return pl.BlockSpec((ROWS_BLK, w), lambda i: (i, 0))


def _part_spec(core, w):
  return pl.BlockSpec((1, ROWS_BLK, w), lambda i, core=core: (core, i, 0))


def _full_spec(r, w):
  return pl.BlockSpec((r, w), lambda i: (0, 0))


def _stage0_kernel(x_ref, wc_ref, b_ref, y_ref, r_ref):
  z = jnp.dot(x_ref[...], wc_ref[...], preferred_element_type=jnp.float32)
  y_ref[...] = z[:, :D]
  r_ref[...] = z[:, D:] + b_ref[...]


def _stage0(x, wc, b):
  return pl.pallas_call(
      _stage0_kernel,
      grid=(GRID,),
      in_specs=[_rows_spec(D), _full_spec(D, 2 * D), _full_spec(1, D)],
      out_specs=[_rows_spec(D), _rows_spec(D)],
      out_shape=[jax.ShapeDtypeStruct((N, D), jnp.float32),
                 jax.ShapeDtypeStruct((N, D), jnp.float32)],
  )(x, wc, b.reshape(1, D))


def _stage1_kernel(p0_ref, p1_ref, r0_ref, wc_ref, b_ref,
                   h_ref, y_ref, r_ref):
  h = jnp.maximum(p0_ref[0] + p1_ref[0] + r0_ref[...], 0.0)
  h_ref[...] = h
  z = jnp.dot(h, wc_ref[...], preferred_element_type=jnp.float32)
  y_ref[...] = z[:, :D]
  r_ref[...] = z[:, D:] + b_ref[...]


def _stage1(parts, r0, wc, b):
  return pl.pallas_call(
      _stage1_kernel,
      grid=(GRID,),
      in_specs=[_part_spec(0, D), _part_spec(1, D), _rows_spec(D),
                _full_spec(D, 2 * D), _full_spec(1, D)],
      out_specs=[_rows_spec(D), _rows_spec(D), _rows_spec(D)],
      out_shape=[jax.ShapeDtypeStruct((N, D), jnp.float32)] * 3,
  )(parts, parts, r0, wc, b.reshape(1, D))


def _stage2_kernel(q0_ref, q1_ref, r1_ref, h0_ref, wc_ref, b_ref,
                   y_ref, r_ref):
  h1 = jnp.maximum(q0_ref[0] + q1_ref[0] + r1_ref[...], 0.0)
  h0 = h0_ref[...]
  norm_prev = jnp.sqrt(jnp.sum(h0 * h0, axis=1, keepdims=True))
  norm_curr = jnp.sqrt(jnp.sum(h1 * h1, axis=1, keepdims=True))
  scaled_prev = h0 * (norm_curr / (norm_prev + 1e-09))
  h = _ALPHA * h1 + (1.0 - _ALPHA) * scaled_prev
  z = jnp.dot(h, wc_ref[...], preferred_element_type=jnp.float32)
  y_ref[...] = z[:, :D2]
  r_ref[...] = z[:, D2:] + b_ref[...]


def _stage2(parts, r1, h0, wc, b):
  return pl.pallas_call(
      _stage2_kernel,
      grid=(GRID,),
      in_specs=[_part_spec(0, D), _part_spec(1, D), _rows_spec(D),
                _rows_spec(D), _full_spec(D, 2 * D2), _full_spec(1, D2)],
      out_specs=[_rows_spec(D2), _rows_spec(D2)],
      out_shape=[jax.ShapeDtypeStruct((N, D2), jnp.float32)] * 2,
  )(parts, parts, r1, h0, wc, b)


def _stage3_kernel(s0_ref, s1_ref, r2_ref, o_ref):
  o_ref[...] = (s0_ref[0] + s1_ref[0] + r2_ref[...])[:, :N_CLS]


def _stage3(parts, r2):
  return pl.pallas_call(
      _stage3_kernel,
      grid=(GRID,),
      in_specs=[_part_spec(0, D2), _part_spec(1, D2), _rows_spec(D2)],
      out_specs=_rows_spec(N_CLS),
      out_shape=jax.ShapeDtypeStruct((N, N_CLS), jnp.float32),
  )(parts, parts, r2)


def kernel(x, edge_index, W_rel0, W_root0, b0, W_rel1, W_root1, b1,
           W_rel2, W_root2, b2):
  pad = E_PAD - E_ORIG
  pad_ar = jnp.arange(pad, dtype=jnp.int32)
  edges_per_tile = CHUNKS_PER_TILE * CHUNK
  tail_start = (2 * N_TILES - 1) * edges_per_tile
  tail = jnp.concatenate(
      [edge_index[:, tail_start:],
       jnp.stack([pad_ar % N, N + pad_ar % (N_PAD - N)])], axis=1)
  zeros128 = jnp.zeros((ROWS_PER_TILE, D), jnp.float32)
  zeros48 = jnp.zeros((ROWS_PER_TILE, D2), jnp.float32)

  wc0 = jnp.concatenate([W_rel0, W_root0], axis=1)
  wc1 = jnp.concatenate([W_rel1, W_root1], axis=1)
  wc2 = jnp.concatenate([jnp.pad(W_rel2, ((0, 0), (0, D2 - N_CLS))),
                         jnp.pad(W_root2, ((0, 0), (0, D2 - N_CLS)))], axis=1)
  b2p = jnp.pad(b2, (0, D2 - N_CLS)).reshape(1, D2)
  y0, root0 = _stage0(x, wc0, b0)
  parts0 = _make_scatter(D)(y0, edge_index, tail, zeros128)
  h0, y1, root1 = _stage1(parts0, root0, wc1, b1)
  parts1 = _make_scatter(D)(y1, edge_index, tail, zeros128)
  y2, root2 = _stage2(parts1, root1, h0, wc2, b2p)
  parts2 = _make_scatter(D2)(y2, edge_index, tail, zeros48)
  return _stage3(parts2, root2)


# speedup vs baseline: 1.0198x; 1.0198x over previous
import functools

import jax
import jax.numpy as jnp
from jax import lax
from jax.experimental import pallas as pl
from jax.experimental.pallas import tpu as pltpu
from jax.experimental.pallas import tpu_sc as plsc

N = 10000
D = 128
N_CLS = 40
D2 = 48
N_PAD = 10240
N_TILES = 16
ROWS_PER_TILE = N_PAD // N_TILES
CHUNK = 112
E_ORIG = 320000
CHUNKS_PER_TILE = 90
E_PAD = 2 * N_TILES * CHUNKS_PER_TILE * CHUNK
EDGES_PER_CORE = E_PAD // 2
NROW = 3
NIDX = 4
ROWS_BLK = 5000
GRID = N // ROWS_BLK


@functools.lru_cache(maxsize=None)
def _make_scatter(d):
  mesh = plsc.VectorSubcoreMesh(core_axis_name="c", subcore_axis_name="s",
                                num_cores=2, num_subcores=N_TILES)

  @functools.partial(
      pl.kernel,
      out_type=jax.ShapeDtypeStruct((2, N_PAD, d), jnp.float32),
      mesh=mesh,
      compiler_params=pltpu.CompilerParams(use_tc_tiling_on_sc=False),
      scratch_types=(
          [pltpu.VMEM((CHUNK, d), jnp.float32)] * NROW +
          [pltpu.VMEM((CHUNK,), jnp.int32)] * NIDX +
          [pltpu.VMEM((CHUNK,), jnp.int32)] * NIDX +
          [pltpu.VMEM_SHARED((N_PAD, d), jnp.float32)] +
          [pltpu.SemaphoreType.DMA] * (2 * NROW + 2 * NIDX)
      ),
  )
  def scatter_kernel(y_hbm, edge_hbm, tail_hbm, zeros_hbm, out_hbm,
                     rb0, rb1, rb2, si0, si1, si2, si3, di0, di1, di2, di3,
                     acc_sh, gs0, gs1, gs2, cs0, cs1, cs2,
                     ss0, ss1, ss2, ss3, ds0, ds1, ds2, ds3):
    rows = (rb0, rb1, rb2)
    sidx = (si0, si1, si2, si3)
    didx = (di0, di1, di2, di3)
    gsem = (gs0, gs1, gs2)
    csem = (cs0, cs1, cs2)
    ssem = (ss0, ss1, ss2, ss3)
    dsem = (ds0, ds1, ds2, ds3)
    c = lax.axis_index("c")
    s = lax.axis_index("s")
    row0 = s * ROWS_PER_TILE
    tile_id = c * N_TILES + s
    base = tile_id * (CHUNKS_PER_TILE * CHUNK)
    is_last = tile_id == (2 * N_TILES - 1)

    def idx_start(j, t):
      e0 = base + j * CHUNK
      o = j * CHUNK

      @pl.when(is_last)
      def _():
        pltpu.async_copy(tail_hbm.at[0, pl.ds(o, CHUNK)], sidx[t], ssem[t])
        pltpu.async_copy(tail_hbm.at[1, pl.ds(o, CHUNK)], didx[t], dsem[t])

      @pl.when(jnp.logical_not(is_last))
      def _():
        pltpu.async_copy(edge_hbm.at[0, pl.ds(e0, CHUNK)], sidx[t], ssem[t])
        pltpu.async_copy(edge_hbm.at[1, pl.ds(e0, CHUNK)], didx[t], dsem[t])

    def idx_wait(t):
      pltpu.make_async_copy(edge_hbm.at[0, pl.ds(0, CHUNK)], sidx[t],
                            ssem[t]).wait()

    def gather_start(b, t):
      pltpu.async_copy(y_hbm.at[sidx[t]], rows[b], gsem[b])

    def gather_wait(b, t):
      pltpu.make_async_copy(y_hbm.at[sidx[t]], rows[b], gsem[b]).wait()

    def scatter_start(b, t):
      pltpu.make_async_copy(edge_hbm.at[1, pl.ds(0, CHUNK)], didx[t],
                            dsem[t]).wait()
      pltpu.async_copy(rows[b], acc_sh.at[didx[t]], csem[b], add=True)

    def scatter_wait(b, t):
      pltpu.make_async_copy(rows[b], acc_sh.at[didx[t]], csem[b]).wait()

    def step(j, jmod, with_swait=True, with_gather=True, with_idx=True):
      b = jmod % NROW
      t = jmod % NIDX
      gather_wait(b, t)
      if with_swait:
        scatter_wait((jmod - 1) % NROW, (jmod - 1) % NIDX)
      if with_gather:
        idx_wait((jmod + 2) % NIDX)
        gather_start((jmod + 2) % NROW, (jmod + 2) % NIDX)
      scatter_start(b, t)
      if with_idx:
        idx_start(j + 3, (jmod + 3) % NIDX)

    idx_start(0, 0)
    idx_start(1, 1)
    idx_start(2, 2)
    idx_wait(0)
    gather_start(0, 0)
    idx_wait(1)
    gather_start(1, 1)
    pltpu.sync_copy(zeros_hbm, acc_sh.at[pl.ds(row0, ROWS_PER_TILE)])
    plsc.subcore_barrier()

    step(0, 0, with_swait=False)

    def body(i, carry):
      for k in range(12):
        step(12 * i + 1 + k, 1 + k)
      return carry

    lax.fori_loop(0, (CHUNKS_PER_TILE - 6) // 12, body, 0)
    for j in range(CHUNKS_PER_TILE - 5, CHUNKS_PER_TILE):
      step(j, j % 12,
           with_gather=(j <= CHUNKS_PER_TILE - 3),
           with_idx=(j <= CHUNKS_PER_TILE - 4))
    scatter_wait((CHUNKS_PER_TILE - 1) % NROW, (CHUNKS_PER_TILE - 1) % NIDX)
    plsc.subcore_barrier()
    pltpu.sync_copy(acc_sh.at[pl.ds(row0, ROWS_PER_TILE)],
                    out_hbm.at[c, pl.ds(row0, ROWS_PER_TILE)])

  return scatter_kernel


_ALPHA = 0.48870956897735596


def _rows_spec(w):
  return pl.BlockSpec((ROWS_BLK, w), lambda i: (i, 0))


def _part_spec(core, w):
  return pl.BlockSpec((1, ROWS_BLK, w), lambda i, core=core: (core, i, 0))


def _full_spec(r, w):
  return pl.BlockSpec((r, w), lambda i: (0, 0))


def _stage0_kernel(x_ref, wc_ref, b_ref, y_ref, r_ref):
  z = jnp.dot(x_ref[...], wc_ref[...], preferred_element_type=jnp.float32)
  y_ref[...] = z[:, :D]
  r_ref[...] = z[:, D:] + b_ref[...]


def _stage0(x, wc, b):
  return pl.pallas_call(
      _stage0_kernel,
      grid=(GRID,),
      in_specs=[_rows_spec(D), _full_spec(D, 2 * D), _full_spec(1, D)],
      out_specs=[_rows_spec(D), _rows_spec(D)],
      out_shape=[jax.ShapeDtypeStruct((N, D), jnp.float32),
                 jax.ShapeDtypeStruct((N, D), jnp.float32)],
  )(x, wc, b.reshape(1, D))


def _stage1_kernel(p0_ref, p1_ref, r0_ref, wc_ref, b_ref,
                   h_ref, y_ref, r_ref):
  h = jnp.maximum(p0_ref[0] + p1_ref[0] + r0_ref[...], 0.0)
  h_ref[...] = h
  z = jnp.dot(h, wc_ref[...], preferred_element_type=jnp.float32)
  y_ref[...] = z[:, :D]
  r_ref[...] = z[:, D:] + b_ref[...]


def _stage1(parts, r0, wc, b):
  return pl.pallas_call(
      _stage1_kernel,
      grid=(GRID,),
      in_specs=[_part_spec(0, D), _part_spec(1, D), _rows_spec(D),
                _full_spec(D, 2 * D), _full_spec(1, D)],
      out_specs=[_rows_spec(D), _rows_spec(D), _rows_spec(D)],
      out_shape=[jax.ShapeDtypeStruct((N, D), jnp.float32)] * 3,
  )(parts, parts, r0, wc, b.reshape(1, D))


def _stage2_kernel(q0_ref, q1_ref, r1_ref, h0_ref, wc_ref, b_ref,
                   y_ref, r_ref):
  h1 = jnp.maximum(q0_ref[0] + q1_ref[0] + r1_ref[...], 0.0)
  h0 = h0_ref[...]
  norm_prev = jnp.sqrt(jnp.sum(h0 * h0, axis=1, keepdims=True))
  norm_curr = jnp.sqrt(jnp.sum(h1 * h1, axis=1, keepdims=True))
  scaled_prev = h0 * (norm_curr / (norm_prev + 1e-09))
  h = _ALPHA * h1 + (1.0 - _ALPHA) * scaled_prev
  z = jnp.dot(h, wc_ref[...], preferred_element_type=jnp.float32)
  y_ref[...] = z[:, :D2]
  r_ref[...] = z[:, D2:] + b_ref[...]


def _stage2(parts, r1, h0, wc, b):
  return pl.pallas_call(
      _stage2_kernel,
      grid=(GRID,),
      in_specs=[_part_spec(0, D), _part_spec(1, D), _rows_spec(D),
                _rows_spec(D), _full_spec(D, 2 * D2), _full_spec(1, D2)],
      out_specs=[_rows_spec(D2), _rows_spec(D2)],
      out_shape=[jax.ShapeDtypeStruct((N, D2), jnp.float32)] * 2,
  )(parts, parts, r1, h0, wc, b)


def _stage3_kernel(s0_ref, s1_ref, r2_ref, o_ref):
  o_ref[...] = (s0_ref[0] + s1_ref[0] + r2_ref[...])[:, :N_CLS]


def _stage3(parts, r2):
  return pl.pallas_call(
      _stage3_kernel,
      grid=(GRID,),
      in_specs=[_part_spec(0, D2), _part_spec(1, D2), _rows_spec(D2)],
      out_specs=_rows_spec(N_CLS),
      out_shape=jax.ShapeDtypeStruct((N, N_CLS), jnp.float32),
  )(parts, parts, r2)


def kernel(x, edge_index, W_rel0, W_root0, b0, W_rel1, W_root1, b1,
           W_rel2, W_root2, b2):
  pad = E_PAD - E_ORIG
  pad_ar = jnp.arange(pad, dtype=jnp.int32)
  edges_per_tile = CHUNKS_PER_TILE * CHUNK
  tail_start = (2 * N_TILES - 1) * edges_per_tile
  tail = jnp.concatenate(
      [edge_index[:, tail_start:],
       jnp.stack([pad_ar % N, N + pad_ar % (N_PAD - N)])], axis=1)
  zeros128 = jnp.zeros((ROWS_PER_TILE, D), jnp.float32)
  zeros48 = jnp.zeros((ROWS_PER_TILE, D2), jnp.float32)

  wc0 = jnp.concatenate([W_rel0, W_root0], axis=1)
  wc1 = jnp.concatenate([W_rel1, W_root1], axis=1)
  wc2 = jnp.concatenate([jnp.pad(W_rel2, ((0, 0), (0, D2 - N_CLS))),
                         jnp.pad(W_root2, ((0, 0), (0, D2 - N_CLS)))], axis=1)
  b2p = jnp.pad(b2, (0, D2 - N_CLS)).reshape(1, D2)
  y0, root0 = _stage0(x, wc0, b0)
  parts0 = _make_scatter(D)(y0, edge_index, tail, zeros128)
  h0, y1, root1 = _stage1(parts0, root0, wc1, b1)
  parts1 = _make_scatter(D)(y1, edge_index, tail, zeros128)
  y2, root2 = _stage2(parts1, root1, h0, wc2, b2p)
  parts2 = _make_scatter(D2)(y2, edge_index, tail, zeros48)
  return _stage3(parts2, root2)
